# baseline (speedup 1.0000x reference)
import functools

import jax
import jax.numpy as jnp
from jax.experimental import pallas as pl
from jax.experimental.pallas import tpu as pltpu
from jax import lax

HIDDEN = 128
NG = 8
NCELLS = NG * NG
GATES = 4 * HIDDEN


def _proj_kernel(h_ref, wp_ref, bp_ref, ot_ref):
    ot_ref[...] = lax.dot_general(h_ref[...], wp_ref[...],
                                  (((1,), (1,)), ((), ())),
                                  preferred_element_type=jnp.float32
                                  ) + bp_ref[...]


def _cell_kernel(x8_ref, gid_ref, sums_ref, wr_w_ref, wr_b_ref, w_ih_ref,
                 w_hh_ref, b_ih_ref, b_hh_ref, c_ref, h2_ref, c2_ref):
    nc = c_ref.shape[0]
    e_t = jnp.maximum(
        jnp.dot(wr_w_ref[...], x8_ref[...],
                preferred_element_type=jnp.float32) + wr_b_ref[...], 0.0)
    oneT = (lax.broadcasted_iota(jnp.int32, (NCELLS, nc), 0)
            == gid_ref[...]).astype(jnp.float32)
    h_soc = lax.dot_general(oneT, sums_ref[...], (((0,), (0,)), ((), ())),
                            precision=lax.Precision.HIGHEST,
                            preferred_element_type=jnp.float32)
    ge = lax.dot_general(e_t, w_ih_ref[...], (((0,), (1,)), ((), ())),
                         preferred_element_type=jnp.float32)
    gs = lax.dot_general(h_soc, w_hh_ref[...], (((1,), (1,)), ((), ())),
                         preferred_element_type=jnp.float32)
    gates = ((ge + b_ih_ref[...]) + gs) + b_hh_ref[...]
    i = jax.nn.sigmoid(gates[:, 0:HIDDEN])
    f = jax.nn.sigmoid(gates[:, HIDDEN:2 * HIDDEN])
    g = jnp.tanh(gates[:, 2 * HIDDEN:3 * HIDDEN])
    o = jax.nn.sigmoid(gates[:, 3 * HIDDEN:4 * HIDDEN])
    c2 = f * c_ref[...] + i * g
    h2_ref[...] = o * jnp.tanh(c2)
    c2_ref[...] = c2


def kernel(x, wr_w, wr_b, w_ih, w_hh, b_ih, b_hh, wp_w, wp_b):
    n, t_steps, _ = x.shape
    nc = min(2048, n)
    assert n % nc == 0
    nchunks = n // nc

    d = 2.0 / NG
    cx = jnp.clip(x[:, :, 0], -1.0, 1.0)
    cy = jnp.clip(x[:, :, 1], -1.0, 1.0)
    xi = jnp.clip(jnp.floor((cx + 1.0) / d).astype(jnp.int32), 0, NG - 1)
    yi = jnp.clip(jnp.floor((cy + 1.0) / d).astype(jnp.int32), 0, NG - 1)
    gid = xi * NG + yi
    gid_t = jnp.transpose(gid)
    gid_rows = gid_t.reshape(t_steps, 1, n)

    xq8 = jnp.pad(jnp.transpose(x, (1, 2, 0)), ((0, 0), (0, 5), (0, 0)))
    wr_w8 = jnp.pad(wr_w, ((0, 0), (0, 5)))
    wr_bc = wr_b.reshape(-1, 1)
    b_ih_r = b_ih.reshape(1, GATES)
    b_hh_r = b_hh.reshape(1, GATES)
    wp8 = jnp.pad(wp_w, ((0, 3), (0, 0)))
    bp8 = jnp.pad(wp_b, (0, 3)).reshape(1, 8)

    step = pl.pallas_call(
        _cell_kernel,
        grid=(nchunks,),
        in_specs=[
            pl.BlockSpec((8, nc), lambda k: (0, k)),
            pl.BlockSpec((1, nc), lambda k: (0, k)),
            pl.BlockSpec((NCELLS, HIDDEN), lambda k: (0, 0)),
            pl.BlockSpec(wr_w8.shape, lambda k: (0, 0)),
            pl.BlockSpec(wr_bc.shape, lambda k: (0, 0)),
            pl.BlockSpec(w_ih.shape, lambda k: (0, 0)),
            pl.BlockSpec(w_hh.shape, lambda k: (0, 0)),
            pl.BlockSpec(b_ih_r.shape, lambda k: (0, 0)),
            pl.BlockSpec(b_hh_r.shape, lambda k: (0, 0)),
            pl.BlockSpec((nc, HIDDEN), lambda k: (k, 0)),
        ],
        out_specs=[
            pl.BlockSpec((nc, HIDDEN), lambda k: (k, 0)),
            pl.BlockSpec((nc, HIDDEN), lambda k: (k, 0)),
        ],
        out_shape=[
            jax.ShapeDtypeStruct((n, HIDDEN), jnp.float32),
            jax.ShapeDtypeStruct((n, HIDDEN), jnp.float32),
        ],
        compiler_params=pltpu.CompilerParams(
            dimension_semantics=("parallel",)),
    )

    proj = pl.pallas_call(
        _proj_kernel,
        grid=(nchunks,),
        in_specs=[
            pl.BlockSpec((nc, HIDDEN), lambda k: (k, 0)),
            pl.BlockSpec(wp8.shape, lambda k: (0, 0)),
            pl.BlockSpec(bp8.shape, lambda k: (0, 0)),
        ],
        out_specs=pl.BlockSpec((nc, 8), lambda k: (k, 0)),
        out_shape=jax.ShapeDtypeStruct((n, 8), jnp.float32),
        compiler_params=pltpu.CompilerParams(
            dimension_semantics=("parallel",)),
    )

    h = jnp.zeros((n, HIDDEN), dtype=x.dtype)
    c = jnp.zeros((n, HIDDEN), dtype=x.dtype)
    out_list = []
    for t in range(t_steps):
        sums = jax.ops.segment_sum(h, gid_t[t], num_segments=NCELLS)
        h, c = step(xq8[t], gid_rows[t], sums, wr_w8, wr_bc, w_ih, w_hh,
                    b_ih_r, b_hh_r, c)
        out_list.append(proj(h, wp8, bp8))

    out = jnp.stack(out_list, axis=0)[:, :, :5]
    out = jnp.transpose(out, (1, 0, 2))
    out = jnp.cumsum(out, axis=2)
    return out, h, c


# speedup vs baseline: 1.1086x; 1.1086x over previous
import functools

import jax
import jax.numpy as jnp
from jax.experimental import pallas as pl
from jax.experimental.pallas import tpu as pltpu
from jax import lax

HIDDEN = 128
NG = 8
NCELLS = NG * NG
GATES = 4 * HIDDEN


def _cell_kernel(x8_ref, gid_ref, sums_ref, wr_w_ref, wr_b_ref, w_ih_ref,
                 w_hh_ref, b_ih_ref, b_hh_ref, c_ref, h2_ref, c2_ref):
    nc = c_ref.shape[0]
    e_t = jnp.maximum(
        jnp.dot(wr_w_ref[...], x8_ref[...],
                preferred_element_type=jnp.float32) + wr_b_ref[...], 0.0)
    oneT = (lax.broadcasted_iota(jnp.int32, (NCELLS, nc), 0)
            == gid_ref[...]).astype(jnp.float32)
    h_soc = lax.dot_general(oneT, sums_ref[...], (((0,), (0,)), ((), ())),
                            precision=lax.Precision.HIGHEST,
                            preferred_element_type=jnp.float32)
    ge = lax.dot_general(e_t, w_ih_ref[...], (((0,), (1,)), ((), ())),
                         preferred_element_type=jnp.float32)
    gs = lax.dot_general(h_soc, w_hh_ref[...], (((1,), (1,)), ((), ())),
                         preferred_element_type=jnp.float32)
    gates = ((ge + b_ih_ref[...]) + gs) + b_hh_ref[...]
    i = jax.nn.sigmoid(gates[:, 0:HIDDEN])
    f = jax.nn.sigmoid(gates[:, HIDDEN:2 * HIDDEN])
    g = jnp.tanh(gates[:, 2 * HIDDEN:3 * HIDDEN])
    o = jax.nn.sigmoid(gates[:, 3 * HIDDEN:4 * HIDDEN])
    c2 = f * c_ref[...] + i * g
    h2_ref[...] = o * jnp.tanh(c2)
    c2_ref[...] = c2


def kernel(x, wr_w, wr_b, w_ih, w_hh, b_ih, b_hh, wp_w, wp_b):
    n, t_steps, _ = x.shape
    nc = min(2048, n)
    assert n % nc == 0
    nchunks = n // nc

    d = 2.0 / NG
    cx = jnp.clip(x[:, :, 0], -1.0, 1.0)
    cy = jnp.clip(x[:, :, 1], -1.0, 1.0)
    xi = jnp.clip(jnp.floor((cx + 1.0) / d).astype(jnp.int32), 0, NG - 1)
    yi = jnp.clip(jnp.floor((cy + 1.0) / d).astype(jnp.int32), 0, NG - 1)
    gid = xi * NG + yi
    gid_t = jnp.transpose(gid)
    gid_rows = gid_t.reshape(t_steps, 1, n)

    xq8 = jnp.pad(jnp.transpose(x, (1, 2, 0)), ((0, 0), (0, 5), (0, 0)))
    wr_w8 = jnp.pad(wr_w, ((0, 0), (0, 5)))
    wr_bc = wr_b.reshape(-1, 1)
    b_ih_r = b_ih.reshape(1, GATES)
    b_hh_r = b_hh.reshape(1, GATES)

    step = pl.pallas_call(
        _cell_kernel,
        grid=(nchunks,),
        in_specs=[
            pl.BlockSpec((8, nc), lambda k: (0, k)),
            pl.BlockSpec((1, nc), lambda k: (0, k)),
            pl.BlockSpec((NCELLS, HIDDEN), lambda k: (0, 0)),
            pl.BlockSpec(wr_w8.shape, lambda k: (0, 0)),
            pl.BlockSpec(wr_bc.shape, lambda k: (0, 0)),
            pl.BlockSpec(w_ih.shape, lambda k: (0, 0)),
            pl.BlockSpec(w_hh.shape, lambda k: (0, 0)),
            pl.BlockSpec(b_ih_r.shape, lambda k: (0, 0)),
            pl.BlockSpec(b_hh_r.shape, lambda k: (0, 0)),
            pl.BlockSpec((nc, HIDDEN), lambda k: (k, 0)),
        ],
        out_specs=[
            pl.BlockSpec((nc, HIDDEN), lambda k: (k, 0)),
            pl.BlockSpec((nc, HIDDEN), lambda k: (k, 0)),
        ],
        out_shape=[
            jax.ShapeDtypeStruct((n, HIDDEN), jnp.float32),
            jax.ShapeDtypeStruct((n, HIDDEN), jnp.float32),
        ],
        compiler_params=pltpu.CompilerParams(
            dimension_semantics=("parallel",)),
    )

    h = jnp.zeros((n, HIDDEN), dtype=x.dtype)
    c = jnp.zeros((n, HIDDEN), dtype=x.dtype)
    hs_list = []
    for t in range(t_steps):
        sums = jax.ops.segment_sum(h, gid_t[t], num_segments=NCELLS)
        h, c = step(xq8[t], gid_rows[t], sums, wr_w8, wr_bc, w_ih, w_hh,
                    b_ih_r, b_hh_r, c)
        hs_list.append(h)

    hs_all = jnp.stack(hs_list, axis=0)
    out = hs_all @ wp_w.T + wp_b
    out = jnp.transpose(out, (1, 0, 2))
    out = jnp.cumsum(out, axis=2)
    return out, h, c


# speedup vs baseline: 1.1937x; 1.0768x over previous
import functools

import jax
import jax.numpy as jnp
from jax.experimental import pallas as pl
from jax.experimental.pallas import tpu as pltpu
from jax import lax

HIDDEN = 128
NG = 8
NCELLS = NG * NG
GATES = 4 * HIDDEN


def _cell_kernel(ge_ref, gid_ref, sums_ref, w_hh_ref, b_hh_ref, c_ref,
                 h2_ref, c2_ref):
    nc = c_ref.shape[0]
    oneT = (lax.broadcasted_iota(jnp.int32, (NCELLS, nc), 0)
            == gid_ref[...]).astype(jnp.float32)
    h_soc = lax.dot_general(oneT, sums_ref[...], (((0,), (0,)), ((), ())),
                            precision=lax.Precision.HIGHEST,
                            preferred_element_type=jnp.float32)
    gs = lax.dot_general(h_soc, w_hh_ref[...], (((1,), (1,)), ((), ())),
                         preferred_element_type=jnp.float32)
    gates = (ge_ref[...] + gs) + b_hh_ref[...]
    i = jax.nn.sigmoid(gates[:, 0:HIDDEN])
    f = jax.nn.sigmoid(gates[:, HIDDEN:2 * HIDDEN])
    g = jnp.tanh(gates[:, 2 * HIDDEN:3 * HIDDEN])
    o = jax.nn.sigmoid(gates[:, 3 * HIDDEN:4 * HIDDEN])
    c2 = f * c_ref[...] + i * g
    h2_ref[...] = o * jnp.tanh(c2)
    c2_ref[...] = c2


def kernel(x, wr_w, wr_b, w_ih, w_hh, b_ih, b_hh, wp_w, wp_b):
    n, t_steps, _ = x.shape
    nc = min(2048, n)
    assert n % nc == 0
    nchunks = n // nc

    d = 2.0 / NG
    cx = jnp.clip(x[:, :, 0], -1.0, 1.0)
    cy = jnp.clip(x[:, :, 1], -1.0, 1.0)
    xi = jnp.clip(jnp.floor((cx + 1.0) / d).astype(jnp.int32), 0, NG - 1)
    yi = jnp.clip(jnp.floor((cy + 1.0) / d).astype(jnp.int32), 0, NG - 1)
    gid = xi * NG + yi
    gid_t = jnp.transpose(gid)
    gid_rows = gid_t.reshape(t_steps, 1, n)

    b_hh_r = b_hh.reshape(1, GATES)

    step = pl.pallas_call(
        _cell_kernel,
        grid=(nchunks,),
        in_specs=[
            pl.BlockSpec((nc, GATES), lambda k: (k, 0)),
            pl.BlockSpec((1, nc), lambda k: (0, k)),
            pl.BlockSpec((NCELLS, HIDDEN), lambda k: (0, 0)),
            pl.BlockSpec(w_hh.shape, lambda k: (0, 0)),
            pl.BlockSpec(b_hh_r.shape, lambda k: (0, 0)),
            pl.BlockSpec((nc, HIDDEN), lambda k: (k, 0)),
        ],
        out_specs=[
            pl.BlockSpec((nc, HIDDEN), lambda k: (k, 0)),
            pl.BlockSpec((nc, HIDDEN), lambda k: (k, 0)),
        ],
        out_shape=[
            jax.ShapeDtypeStruct((n, HIDDEN), jnp.float32),
            jax.ShapeDtypeStruct((n, HIDDEN), jnp.float32),
        ],
        compiler_params=pltpu.CompilerParams(
            dimension_semantics=("parallel",)),
    )

    embed = jax.nn.relu(x @ wr_w.T + wr_b)

    h = jnp.zeros((n, HIDDEN), dtype=x.dtype)
    c = jnp.zeros((n, HIDDEN), dtype=x.dtype)
    out_list = []
    for t in range(t_steps):
        sums = jax.ops.segment_sum(h, gid_t[t], num_segments=NCELLS)
        ge = embed[:, t, :] @ w_ih.T + b_ih
        h, c = step(ge, gid_rows[t], sums, w_hh, b_hh_r, c)
        out_list.append(h @ wp_w.T + wp_b)

    out = jnp.stack(out_list, axis=0)
    out = jnp.transpose(out, (1, 0, 2))
    out = jnp.cumsum(out, axis=2)
    return out, h, c


# speedup vs baseline: 1.2146x; 1.0175x over previous
import functools

import jax
import jax.numpy as jnp
from jax.experimental import pallas as pl
from jax.experimental.pallas import tpu as pltpu
from jax import lax

HIDDEN = 128
NG = 8
NCELLS = NG * NG
GATES = 4 * HIDDEN


def _cell_kernel(ge_ref, gid_ref, sums_ref, w_hh_ref, b_hh_ref, c_ref,
                 h2_ref, c2_ref):
    nc = c_ref.shape[0]
    oneT = (lax.broadcasted_iota(jnp.int32, (NCELLS, nc), 0)
            == gid_ref[...]).astype(jnp.float32)
    h_soc = lax.dot_general(oneT, sums_ref[...], (((0,), (0,)), ((), ())),
                            precision=lax.Precision.HIGHEST,
                            preferred_element_type=jnp.float32)
    gs = lax.dot_general(h_soc, w_hh_ref[...], (((1,), (1,)), ((), ())),
                         preferred_element_type=jnp.float32)
    gates = (ge_ref[...] + gs) + b_hh_ref[...]
    i = jax.nn.sigmoid(gates[:, 0:HIDDEN])
    f = jax.nn.sigmoid(gates[:, HIDDEN:2 * HIDDEN])
    g = jnp.tanh(gates[:, 2 * HIDDEN:3 * HIDDEN])
    o = jax.nn.sigmoid(gates[:, 3 * HIDDEN:4 * HIDDEN])
    c2 = f * c_ref[...] + i * g
    h2_ref[...] = o * jnp.tanh(c2)
    c2_ref[...] = c2


def kernel(x, wr_w, wr_b, w_ih, w_hh, b_ih, b_hh, wp_w, wp_b):
    n, t_steps, _ = x.shape
    nc = min(2048, n)
    assert n % nc == 0
    nchunks = n // nc

    d = 2.0 / NG
    cx = jnp.clip(x[:, :, 0], -1.0, 1.0)
    cy = jnp.clip(x[:, :, 1], -1.0, 1.0)
    xi = jnp.clip(jnp.floor((cx + 1.0) / d).astype(jnp.int32), 0, NG - 1)
    yi = jnp.clip(jnp.floor((cy + 1.0) / d).astype(jnp.int32), 0, NG - 1)
    gid = xi * NG + yi
    gid_t = jnp.transpose(gid)
    gid_rows = gid_t.reshape(t_steps, 1, n)

    b_hh_r = b_hh.reshape(1, GATES)

    step = pl.pallas_call(
        _cell_kernel,
        grid=(nchunks,),
        in_specs=[
            pl.BlockSpec((nc, GATES), lambda k: (k, 0)),
            pl.BlockSpec((1, nc), lambda k: (0, k)),
            pl.BlockSpec((NCELLS, HIDDEN), lambda k: (0, 0)),
            pl.BlockSpec(w_hh.shape, lambda k: (0, 0)),
            pl.BlockSpec(b_hh_r.shape, lambda k: (0, 0)),
            pl.BlockSpec((nc, HIDDEN), lambda k: (k, 0)),
        ],
        out_specs=[
            pl.BlockSpec((nc, HIDDEN), lambda k: (k, 0)),
            pl.BlockSpec((nc, HIDDEN), lambda k: (k, 0)),
        ],
        out_shape=[
            jax.ShapeDtypeStruct((n, HIDDEN), jnp.float32),
            jax.ShapeDtypeStruct((n, HIDDEN), jnp.float32),
        ],
        compiler_params=pltpu.CompilerParams(
            dimension_semantics=("parallel",)),
    )

    embed = jax.nn.relu(x @ wr_w.T + wr_b)

    h = jnp.zeros((n, HIDDEN), dtype=x.dtype)
    c = jnp.zeros((n, HIDDEN), dtype=x.dtype)
    out_list = []
    for t in range(t_steps):
        if t == 0:
            sums = jnp.zeros((NCELLS, HIDDEN), dtype=x.dtype)
        else:
            sums = jax.ops.segment_sum(h, gid_t[t], num_segments=NCELLS)
        ge = embed[:, t, :] @ w_ih.T + b_ih
        h, c = step(ge, gid_rows[t], sums, w_hh, b_hh_r, c)
        out_list.append(h @ wp_w.T + wp_b)

    out = jnp.stack(out_list, axis=0)
    out = jnp.transpose(out, (1, 0, 2))
    out = jnp.cumsum(out, axis=2)
    return out, h, c


# speedup vs baseline: 1.2240x; 1.0077x over previous
import functools

import jax
import jax.numpy as jnp
from jax.experimental import pallas as pl
from jax.experimental.pallas import tpu as pltpu
from jax import lax

HIDDEN = 128
NG = 8
NCELLS = NG * NG
GATES = 4 * HIDDEN


def _cell_kernel(ge_ref, gid_ref, sums_ref, w_hh_ref, b_hh_ref, c_ref,
                 h2_ref, c2_ref):
    nc = c_ref.shape[0]
    oneT = (lax.broadcasted_iota(jnp.int32, (NCELLS, nc), 0)
            == gid_ref[...]).astype(jnp.float32)
    h_soc = lax.dot_general(oneT, sums_ref[...], (((0,), (0,)), ((), ())),
                            precision=lax.Precision.HIGHEST,
                            preferred_element_type=jnp.float32)
    gs = lax.dot_general(h_soc, w_hh_ref[...], (((1,), (1,)), ((), ())),
                         preferred_element_type=jnp.float32)
    gates = (ge_ref[...] + gs) + b_hh_ref[...]
    i = jax.nn.sigmoid(gates[:, 0:HIDDEN])
    f = jax.nn.sigmoid(gates[:, HIDDEN:2 * HIDDEN])
    g = jnp.tanh(gates[:, 2 * HIDDEN:3 * HIDDEN])
    o = jax.nn.sigmoid(gates[:, 3 * HIDDEN:4 * HIDDEN])
    c2 = f * c_ref[...] + i * g
    h2_ref[...] = o * jnp.tanh(c2)
    c2_ref[...] = c2


def kernel(x, wr_w, wr_b, w_ih, w_hh, b_ih, b_hh, wp_w, wp_b):
    n, t_steps, _ = x.shape
    nc = min(4096, n)
    assert n % nc == 0
    nchunks = n // nc

    d = 2.0 / NG
    cx = jnp.clip(x[:, :, 0], -1.0, 1.0)
    cy = jnp.clip(x[:, :, 1], -1.0, 1.0)
    xi = jnp.clip(jnp.floor((cx + 1.0) / d).astype(jnp.int32), 0, NG - 1)
    yi = jnp.clip(jnp.floor((cy + 1.0) / d).astype(jnp.int32), 0, NG - 1)
    gid = xi * NG + yi
    gid_t = jnp.transpose(gid)
    gid_rows = gid_t.reshape(t_steps, 1, n)

    b_hh_r = b_hh.reshape(1, GATES)

    step = pl.pallas_call(
        _cell_kernel,
        grid=(nchunks,),
        in_specs=[
            pl.BlockSpec((nc, GATES), lambda k: (k, 0)),
            pl.BlockSpec((1, nc), lambda k: (0, k)),
            pl.BlockSpec((NCELLS, HIDDEN), lambda k: (0, 0)),
            pl.BlockSpec(w_hh.shape, lambda k: (0, 0)),
            pl.BlockSpec(b_hh_r.shape, lambda k: (0, 0)),
            pl.BlockSpec((nc, HIDDEN), lambda k: (k, 0)),
        ],
        out_specs=[
            pl.BlockSpec((nc, HIDDEN), lambda k: (k, 0)),
            pl.BlockSpec((nc, HIDDEN), lambda k: (k, 0)),
        ],
        out_shape=[
            jax.ShapeDtypeStruct((n, HIDDEN), jnp.float32),
            jax.ShapeDtypeStruct((n, HIDDEN), jnp.float32),
        ],
        compiler_params=pltpu.CompilerParams(
            dimension_semantics=("parallel",)),
    )

    embed = jax.nn.relu(x @ wr_w.T + wr_b)

    h = jnp.zeros((n, HIDDEN), dtype=x.dtype)
    c = jnp.zeros((n, HIDDEN), dtype=x.dtype)
    out_list = []
    for t in range(t_steps):
        if t == 0:
            sums = jnp.zeros((NCELLS, HIDDEN), dtype=x.dtype)
        else:
            sums = jax.ops.segment_sum(h, gid_t[t], num_segments=NCELLS)
        ge = embed[:, t, :] @ w_ih.T + b_ih
        h, c = step(ge, gid_rows[t], sums, w_hh, b_hh_r, c)
        out_list.append(h @ wp_w.T + wp_b)

    out = jnp.stack(out_list, axis=0)
    out = jnp.transpose(out, (1, 0, 2))
    out = jnp.cumsum(out, axis=2)
    return out, h, c
